# baseline (speedup 1.0000x reference)
import jax
import jax.numpy as jnp
from jax.experimental import pallas as pl
from jax.experimental.pallas import tpu as pltpu

B = 64
NPG = 1536
N = B * NPG
D = 256
C = 512
NB = 1024
TILE = 768
BLOCKS_PER_GRAPH = NPG // TILE


def _syn_kernel(bit_probs_ref, ht_ref, wpt_ref, bp_ref, w1bt_ref, b1_ref, out_ref):
    p = bit_probs_ref[:]
    llr = jnp.log((p + 1e-08) / (1.0 - p + 1e-08))
    llr = jnp.clip(llr, -10.0, 10.0)
    th = jnp.tanh(0.5 * llr)
    soft_syn = jnp.tanh(0.5 * jnp.dot(th, ht_ref[:], preferred_element_type=jnp.float32))
    prob = 0.5 * (1.0 - soft_syn)
    syn_feat = jnp.dot(prob, wpt_ref[:], preferred_element_type=jnp.float32) + bp_ref[:]
    out_ref[:] = jnp.dot(syn_feat, w1bt_ref[:], preferred_element_type=jnp.float32) + b1_ref[:]


def _mlp_kernel(x_ref, syn_ref, mask_ref, w1at_ref, w2t_ref, gamma_ref, beta_ref,
                b2_ref, sw_ref, out_ref):
    i = pl.program_id(0)
    g = i // BLOCKS_PER_GRAPH
    x = x_ref[:]
    h = jnp.dot(x, w1at_ref[:], preferred_element_type=jnp.float32)
    h = h + syn_ref[pl.ds(g, 1), :]
    mu = jnp.mean(h, axis=-1, keepdims=True)
    var = jnp.mean((h - mu) ** 2, axis=-1, keepdims=True)
    h = (h - mu) * jax.lax.rsqrt(var + 1e-05) * gamma_ref[:] + beta_ref[:]
    h = jnp.maximum(h, 0.0)
    enhanced = jnp.dot(h, w2t_ref[:], preferred_element_type=jnp.float32) + b2_ref[:]
    sw = sw_ref[0]
    out_ref[:] = x + mask_ref[:] * (sw * (enhanced - x))


def kernel(node_features, bit_probs, H, var_node_mask, Wp, bp, W1, b1, gamma, beta, W2, b2, syndrome_weight):
    syn_contrib = pl.pallas_call(
        _syn_kernel,
        out_shape=jax.ShapeDtypeStruct((B, D), jnp.float32),
    )(
        bit_probs,
        H.T,
        Wp.T,
        bp.reshape(1, D // 2),
        W1[:, D:].T,
        b1.reshape(1, D),
    )

    mask_f = var_node_mask.astype(jnp.float32).reshape(N, 1)
    grid = (N // TILE,)
    out = pl.pallas_call(
        _mlp_kernel,
        grid=grid,
        in_specs=[
            pl.BlockSpec((TILE, D), lambda i: (i, 0)),
            pl.BlockSpec((B, D), lambda i: (0, 0)),
            pl.BlockSpec((TILE, 1), lambda i: (i, 0)),
            pl.BlockSpec((D, D), lambda i: (0, 0)),
            pl.BlockSpec((D, D), lambda i: (0, 0)),
            pl.BlockSpec((1, D), lambda i: (0, 0)),
            pl.BlockSpec((1, D), lambda i: (0, 0)),
            pl.BlockSpec((1, D), lambda i: (0, 0)),
            pl.BlockSpec(memory_space=pltpu.SMEM),
        ],
        out_specs=pl.BlockSpec((TILE, D), lambda i: (i, 0)),
        out_shape=jax.ShapeDtypeStruct((N, D), jnp.float32),
        compiler_params=pltpu.CompilerParams(
            dimension_semantics=("parallel",),
        ),
    )(
        node_features,
        syn_contrib,
        mask_f,
        W1[:, :D].T,
        W2.T,
        gamma.reshape(1, D),
        beta.reshape(1, D),
        b2.reshape(1, D),
        syndrome_weight.reshape(1),
    )
    return out


# speedup vs baseline: 1.3392x; 1.3392x over previous
import jax
import jax.numpy as jnp
from jax.experimental import pallas as pl
from jax.experimental.pallas import tpu as pltpu

B = 64
NPG = 1536
N = B * NPG
D = 256
C = 512
NB = 1024
TILE = 3072
GRAPHS_PER_TILE = TILE // NPG


def _syn_kernel(bit_probs_ref, ht_ref, wpt_ref, bp_ref, w1bt_ref, b1_ref, out_ref):
    p = bit_probs_ref[:]
    llr = jnp.log((p + 1e-08) / (1.0 - p + 1e-08))
    llr = jnp.clip(llr, -10.0, 10.0)
    th = jnp.tanh(0.5 * llr)
    soft_syn = jnp.tanh(0.5 * jnp.dot(th, ht_ref[:], preferred_element_type=jnp.float32))
    prob = 0.5 * (1.0 - soft_syn)
    syn_feat = jnp.dot(prob, wpt_ref[:], preferred_element_type=jnp.float32) + bp_ref[:]
    out_ref[:] = jnp.dot(syn_feat, w1bt_ref[:], preferred_element_type=jnp.float32) + b1_ref[:]


def _mlp_kernel(x_ref, syn_ref, mask_ref, w1at_ref, w2t_ref, gamma_ref, beta_ref,
                b2_ref, sw_ref, out_ref):
    x = x_ref[:]
    h = jnp.dot(x, w1at_ref[:], preferred_element_type=jnp.float32)
    rows = jax.lax.broadcasted_iota(jnp.int32, (TILE, 1), 0)
    syn_tile = jnp.where(rows < NPG, syn_ref[0, 0:1, :], syn_ref[0, 1:2, :])
    h = h + syn_tile
    mu = jnp.mean(h, axis=-1, keepdims=True)
    var = jnp.mean((h - mu) ** 2, axis=-1, keepdims=True)
    h = (h - mu) * jax.lax.rsqrt(var + 1e-05) * gamma_ref[:] + beta_ref[:]
    h = jnp.maximum(h, 0.0)
    enhanced = jnp.dot(h, w2t_ref[:], preferred_element_type=jnp.float32) + b2_ref[:]
    sw = sw_ref[0]
    out_ref[:] = x + mask_ref[:] * (sw * (enhanced - x))


def kernel(node_features, bit_probs, H, var_node_mask, Wp, bp, W1, b1, gamma, beta, W2, b2, syndrome_weight):
    syn_contrib = pl.pallas_call(
        _syn_kernel,
        out_shape=jax.ShapeDtypeStruct((B, D), jnp.float32),
    )(
        bit_probs,
        H.T,
        Wp.T,
        bp.reshape(1, D // 2),
        W1[:, D:].T,
        b1.reshape(1, D),
    )

    mask_f = var_node_mask.astype(jnp.float32).reshape(N, 1)
    grid = (N // TILE,)
    out = pl.pallas_call(
        _mlp_kernel,
        grid=grid,
        in_specs=[
            pl.BlockSpec((TILE, D), lambda i: (i, 0)),
            pl.BlockSpec((1, GRAPHS_PER_TILE, D), lambda i: (i, 0, 0)),
            pl.BlockSpec((TILE, 1), lambda i: (i, 0)),
            pl.BlockSpec((D, D), lambda i: (0, 0)),
            pl.BlockSpec((D, D), lambda i: (0, 0)),
            pl.BlockSpec((1, D), lambda i: (0, 0)),
            pl.BlockSpec((1, D), lambda i: (0, 0)),
            pl.BlockSpec((1, D), lambda i: (0, 0)),
            pl.BlockSpec(memory_space=pltpu.SMEM),
        ],
        out_specs=pl.BlockSpec((TILE, D), lambda i: (i, 0)),
        out_shape=jax.ShapeDtypeStruct((N, D), jnp.float32),
        compiler_params=pltpu.CompilerParams(
            dimension_semantics=("parallel",),
        ),
    )(
        node_features,
        syn_contrib.reshape(B // GRAPHS_PER_TILE, GRAPHS_PER_TILE, D),
        mask_f,
        W1[:, :D].T,
        W2.T,
        gamma.reshape(1, D),
        beta.reshape(1, D),
        b2.reshape(1, D),
        syndrome_weight.reshape(1),
    )
    return out


# speedup vs baseline: 1.5194x; 1.1345x over previous
import jax
import jax.numpy as jnp
from jax.experimental import pallas as pl
from jax.experimental.pallas import tpu as pltpu

B = 64
NPG = 1536
N = B * NPG
D = 256
C = 512
NB = 1024
TILE = 6144
GRAPHS_PER_TILE = TILE // NPG


def _syn_kernel(bit_probs_ref, ht_ref, wpt_ref, bp_ref, w1bt_ref, b1_ref, out_ref):
    p = bit_probs_ref[:]
    llr = jnp.log((p + 1e-08) / (1.0 - p + 1e-08))
    llr = jnp.clip(llr, -10.0, 10.0)
    th = jnp.tanh(0.5 * llr)
    soft_syn = jnp.tanh(0.5 * jnp.dot(th, ht_ref[:], preferred_element_type=jnp.float32))
    prob = 0.5 * (1.0 - soft_syn)
    syn_feat = jnp.dot(prob, wpt_ref[:], preferred_element_type=jnp.float32) + bp_ref[:]
    out_ref[:] = jnp.dot(syn_feat, w1bt_ref[:], preferred_element_type=jnp.float32) + b1_ref[:]


def _mlp_kernel(x_ref, syn_ref, mask_ref, w1at_ref, w2t_ref, gamma_ref, beta_ref,
                b2_ref, sw_ref, out_ref):
    x = x_ref[:]
    h = jnp.dot(x, w1at_ref[:], preferred_element_type=jnp.float32)
    rows = jax.lax.broadcasted_iota(jnp.int32, (TILE, GRAPHS_PER_TILE), 0)
    cols = jax.lax.broadcasted_iota(jnp.int32, (TILE, GRAPHS_PER_TILE), 1)
    onehot = (rows // NPG == cols).astype(jnp.float32)
    h = h + jnp.dot(onehot, syn_ref[0], preferred_element_type=jnp.float32)
    mu = jnp.mean(h, axis=-1, keepdims=True)
    var = jnp.mean((h - mu) ** 2, axis=-1, keepdims=True)
    h = (h - mu) * jax.lax.rsqrt(var + 1e-05) * gamma_ref[:] + beta_ref[:]
    h = jnp.maximum(h, 0.0)
    enhanced = jnp.dot(h, w2t_ref[:], preferred_element_type=jnp.float32) + b2_ref[:]
    sw = sw_ref[0]
    out_ref[:] = x + mask_ref[:] * (sw * (enhanced - x))


def kernel(node_features, bit_probs, H, var_node_mask, Wp, bp, W1, b1, gamma, beta, W2, b2, syndrome_weight):
    syn_contrib = pl.pallas_call(
        _syn_kernel,
        out_shape=jax.ShapeDtypeStruct((B, D), jnp.float32),
    )(
        bit_probs,
        H.T,
        Wp.T,
        bp.reshape(1, D // 2),
        W1[:, D:].T,
        b1.reshape(1, D),
    )

    mask_f = var_node_mask.astype(jnp.float32).reshape(N, 1)
    grid = (N // TILE,)
    out = pl.pallas_call(
        _mlp_kernel,
        grid=grid,
        in_specs=[
            pl.BlockSpec((TILE, D), lambda i: (i, 0)),
            pl.BlockSpec((1, GRAPHS_PER_TILE, D), lambda i: (i, 0, 0)),
            pl.BlockSpec((TILE, 1), lambda i: (i, 0)),
            pl.BlockSpec((D, D), lambda i: (0, 0)),
            pl.BlockSpec((D, D), lambda i: (0, 0)),
            pl.BlockSpec((1, D), lambda i: (0, 0)),
            pl.BlockSpec((1, D), lambda i: (0, 0)),
            pl.BlockSpec((1, D), lambda i: (0, 0)),
            pl.BlockSpec(memory_space=pltpu.SMEM),
        ],
        out_specs=pl.BlockSpec((TILE, D), lambda i: (i, 0)),
        out_shape=jax.ShapeDtypeStruct((N, D), jnp.float32),
        compiler_params=pltpu.CompilerParams(
            dimension_semantics=("parallel",),
        ),
    )(
        node_features,
        syn_contrib.reshape(B // GRAPHS_PER_TILE, GRAPHS_PER_TILE, D),
        mask_f,
        W1[:, :D].T,
        W2.T,
        gamma.reshape(1, D),
        beta.reshape(1, D),
        b2.reshape(1, D),
        syndrome_weight.reshape(1),
    )
    return out


# speedup vs baseline: 1.5495x; 1.0199x over previous
import jax
import jax.numpy as jnp
from jax.experimental import pallas as pl
from jax.experimental.pallas import tpu as pltpu

B = 64
NPG = 1536
N = B * NPG
D = 256
C = 512
NB = 1024
TILE = 6144
GRAPHS_PER_TILE = TILE // NPG


def _kernel(bit_probs_ref, ht_ref, wpt_ref, bp_ref, w1bt_ref, b1_ref,
            x_ref, msw_ref, w1at_ref, w2t_ref, gamma_ref, beta_ref, b2_ref,
            out_ref, syn_ref):
    i = pl.program_id(0)

    @pl.when(i == 0)
    def _compute_syn():
        p = bit_probs_ref[:]
        llr = jnp.log((p + 1e-08) / (1.0 - p + 1e-08))
        llr = jnp.clip(llr, -10.0, 10.0)
        th = jnp.tanh(0.5 * llr)
        soft_syn = jnp.tanh(0.5 * jnp.dot(th, ht_ref[:], preferred_element_type=jnp.float32))
        prob = 0.5 * (1.0 - soft_syn)
        syn_feat = jnp.dot(prob, wpt_ref[:], preferred_element_type=jnp.float32) + bp_ref[:]
        syn_ref[:] = jnp.dot(syn_feat, w1bt_ref[:], preferred_element_type=jnp.float32) + b1_ref[:]

    x = x_ref[:]
    h = jnp.dot(x, w1at_ref[:], preferred_element_type=jnp.float32)
    rows = jax.lax.broadcasted_iota(jnp.int32, (TILE, B), 0)
    cols = jax.lax.broadcasted_iota(jnp.int32, (TILE, B), 1)
    onehot = (rows // NPG + i * GRAPHS_PER_TILE == cols).astype(jnp.float32)
    h = h + jnp.dot(onehot, syn_ref[:], preferred_element_type=jnp.float32)
    mu = jnp.mean(h, axis=-1, keepdims=True)
    var = jnp.mean((h - mu) ** 2, axis=-1, keepdims=True)
    h = (h - mu) * jax.lax.rsqrt(var + 1e-05) * gamma_ref[:] + beta_ref[:]
    h = jnp.maximum(h, 0.0)
    enhanced = jnp.dot(h, w2t_ref[:], preferred_element_type=jnp.float32) + b2_ref[:]
    out_ref[:] = x + msw_ref[:] * (enhanced - x)


def kernel(node_features, bit_probs, H, var_node_mask, Wp, bp, W1, b1, gamma, beta, W2, b2, syndrome_weight):
    msw = var_node_mask.astype(jnp.float32).reshape(N, 1) * syndrome_weight
    grid = (N // TILE,)
    full = lambda i: (0, 0)
    out = pl.pallas_call(
        _kernel,
        grid=grid,
        in_specs=[
            pl.BlockSpec((B, NB), full),
            pl.BlockSpec((NB, C), full),
            pl.BlockSpec((C, D // 2), full),
            pl.BlockSpec((1, D // 2), full),
            pl.BlockSpec((D // 2, D), full),
            pl.BlockSpec((1, D), full),
            pl.BlockSpec((TILE, D), lambda i: (i, 0)),
            pl.BlockSpec((TILE, 1), lambda i: (i, 0)),
            pl.BlockSpec((D, D), full),
            pl.BlockSpec((D, D), full),
            pl.BlockSpec((1, D), full),
            pl.BlockSpec((1, D), full),
            pl.BlockSpec((1, D), full),
        ],
        out_specs=pl.BlockSpec((TILE, D), lambda i: (i, 0)),
        out_shape=jax.ShapeDtypeStruct((N, D), jnp.float32),
        scratch_shapes=[pltpu.VMEM((B, D), jnp.float32)],
        compiler_params=pltpu.CompilerParams(
            dimension_semantics=("arbitrary",),
        ),
    )(
        bit_probs,
        H.T,
        Wp.T,
        bp.reshape(1, D // 2),
        W1[:, D:].T,
        b1.reshape(1, D),
        node_features,
        msw,
        W1[:, :D].T,
        W2.T,
        gamma.reshape(1, D),
        beta.reshape(1, D),
        b2.reshape(1, D),
    )
    return out
